# Initial kernel scaffold; baseline (speedup 1.0000x reference)
import jax
import jax.numpy as jnp
from jax.experimental import pallas as pl


def kernel(x, edge_index, W1, b1, W2, b2, Wh, bh):
    raise NotImplementedError("write your pallas kernel here")


# speedup vs baseline: 19.5554x; 19.5554x over previous
import functools

import jax
import jax.numpy as jnp
from jax import lax
from jax.experimental import pallas as pl
from jax.experimental.pallas import tpu as pltpu
from jax.experimental.pallas import tpu_sc as plsc

N_NODES = 10000
N_PAD = 10240
D = 128
NC, NS = 2, 16
NW = NC * NS
CH = 128
RPS = N_PAD // NS
MM_BLK = 1024


def _vector_mesh():
    return plsc.VectorSubcoreMesh(
        core_axis_name="c", subcore_axis_name="s", num_cores=NC)


def _sc_degree_hist(dst_idx, ones_blk, zeros_hist):
    chunks = dst_idx.shape[1]

    @functools.partial(
        pl.kernel,
        out_type=jax.ShapeDtypeStruct((NC, N_PAD, D), jnp.float32),
        mesh=_vector_mesh(),
        scratch_types=[
            pltpu.VMEM((chunks, CH), jnp.int32),
            pltpu.VMEM((CH, D), jnp.float32),
            pltpu.VMEM_SHARED((N_PAD, D), jnp.float32),
        ],
    )
    def hist_kernel(dst_hbm, ones_hbm, zeros_hbm, out_hbm,
                    idx_v, ones_v, hist_sh):
        cid = lax.axis_index("c")
        sid = lax.axis_index("s")
        wid = cid * NS + sid
        pltpu.sync_copy(dst_hbm.at[wid], idx_v)
        pltpu.sync_copy(ones_hbm, ones_v)

        @pl.when(sid == 0)
        def _():
            pltpu.sync_copy(zeros_hbm, hist_sh)

        plsc.subcore_barrier()

        @pl.loop(0, chunks)
        def _(j):
            pltpu.sync_copy(ones_v, hist_sh.at[idx_v.at[j]], add=True)

        plsc.subcore_barrier()
        for s in range(NS):
            @pl.when(sid == s)
            def _():
                pltpu.sync_copy(hist_sh.at[pl.ds(s * RPS, RPS)],
                                out_hbm.at[cid, pl.ds(s * RPS, RPS)])

    return hist_kernel(dst_idx, ones_blk, zeros_hist)


def _sc_gather_scatter(g, src_idx, dst_idx, zeros_acc):
    chunks = src_idx.shape[1]

    @functools.partial(
        pl.kernel,
        out_type=jax.ShapeDtypeStruct((NC, N_PAD, D), jnp.float32),
        mesh=_vector_mesh(),
        scratch_types=[
            pltpu.VMEM((chunks, CH), jnp.int32),
            pltpu.VMEM((chunks, CH), jnp.int32),
            pltpu.VMEM((CH, D), jnp.float32),
            pltpu.VMEM_SHARED((N_PAD, D), jnp.float32),
            pltpu.SemaphoreType.DMA,
        ],
    )
    def agg_kernel(g_hbm, src_hbm, dst_hbm, zeros_hbm, out_hbm,
                   src_v, dst_v, buf, acc_sh, sem):
        cid = lax.axis_index("c")
        sid = lax.axis_index("s")
        wid = cid * NS + sid
        pltpu.sync_copy(src_hbm.at[wid], src_v)
        pltpu.sync_copy(dst_hbm.at[wid], dst_v)

        @pl.when(sid == 0)
        def _():
            pltpu.sync_copy(zeros_hbm, acc_sh)

        plsc.subcore_barrier()

        @pl.loop(0, chunks)
        def _(j):
            pltpu.async_copy(g_hbm.at[src_v.at[j]], buf, sem).wait()
            pltpu.sync_copy(buf, acc_sh.at[dst_v.at[j]], add=True)

        plsc.subcore_barrier()
        for s in range(NS):
            @pl.when(sid == s)
            def _():
                pltpu.sync_copy(acc_sh.at[pl.ds(s * RPS, RPS)],
                                out_hbm.at[cid, pl.ds(s * RPS, RPS)])

    return agg_kernel(g, src_idx, dst_idx, zeros_acc)


def _dot(a, b):
    return lax.dot_general(a, b, (((1,), (0,)), ((), ())),
                           preferred_element_type=jnp.float32,
                           precision=lax.Precision.DEFAULT)


def _tc_matmul(x, W):
    def body(x_ref, w_ref, o_ref):
        o_ref[...] = _dot(x_ref[...], w_ref[...])

    return pl.pallas_call(
        body,
        grid=(N_PAD // MM_BLK,),
        in_specs=[pl.BlockSpec((MM_BLK, D), lambda i: (i, 0)),
                  pl.BlockSpec((D, D), lambda i: (0, 0))],
        out_specs=pl.BlockSpec((MM_BLK, D), lambda i: (i, 0)),
        out_shape=jax.ShapeDtypeStruct((N_PAD, D), jnp.float32),
    )(x, W)


def _tc_norm_scale(hist, p):
    def body(h_ref, p_ref, g_ref, d_ref):
        deg = h_ref[0, :, 0:1] + h_ref[1, :, 0:1] + 1.0
        dis = lax.rsqrt(deg)
        g_ref[...] = p_ref[...] * dis
        d_ref[...] = dis

    return pl.pallas_call(
        body,
        grid=(N_PAD // MM_BLK,),
        in_specs=[pl.BlockSpec((NC, MM_BLK, D), lambda i: (0, i, 0)),
                  pl.BlockSpec((MM_BLK, D), lambda i: (i, 0))],
        out_specs=[pl.BlockSpec((MM_BLK, D), lambda i: (i, 0)),
                   pl.BlockSpec((MM_BLK, 1), lambda i: (i, 0))],
        out_shape=[jax.ShapeDtypeStruct((N_PAD, D), jnp.float32),
                   jax.ShapeDtypeStruct((N_PAD, 1), jnp.float32)],
    )(hist, p)


def _tc_layer(acc, g, dis, b, W):
    def body(a_ref, g_ref, d_ref, b_ref, w_ref, o_ref):
        s = a_ref[0] + a_ref[1] + g_ref[...]
        h = jnp.maximum(s * d_ref[...] + b_ref[...], 0.0)
        o_ref[...] = _dot(h, w_ref[...]) * d_ref[...]

    return pl.pallas_call(
        body,
        grid=(N_PAD // MM_BLK,),
        in_specs=[pl.BlockSpec((NC, MM_BLK, D), lambda i: (0, i, 0)),
                  pl.BlockSpec((MM_BLK, D), lambda i: (i, 0)),
                  pl.BlockSpec((MM_BLK, 1), lambda i: (i, 0)),
                  pl.BlockSpec((1, D), lambda i: (0, 0)),
                  pl.BlockSpec((D, D), lambda i: (0, 0))],
        out_specs=pl.BlockSpec((MM_BLK, D), lambda i: (i, 0)),
        out_shape=jax.ShapeDtypeStruct((N_PAD, D), jnp.float32),
    )(acc, g, dis, b, W)


def _tc_final(acc, g, dis, b, wh, bh):
    def body(a_ref, g_ref, d_ref, b_ref, w_ref, bh_ref, o_ref):
        s = a_ref[0] + a_ref[1] + g_ref[...]
        h = jnp.maximum(s * d_ref[...] + b_ref[...], 0.0)
        o_ref[...] = _dot(h, w_ref[...]) + bh_ref[...]

    return pl.pallas_call(
        body,
        grid=(N_PAD // MM_BLK,),
        in_specs=[pl.BlockSpec((NC, MM_BLK, D), lambda i: (0, i, 0)),
                  pl.BlockSpec((MM_BLK, D), lambda i: (i, 0)),
                  pl.BlockSpec((MM_BLK, 1), lambda i: (i, 0)),
                  pl.BlockSpec((1, D), lambda i: (0, 0)),
                  pl.BlockSpec((D, 1), lambda i: (0, 0)),
                  pl.BlockSpec((1, 1), lambda i: (0, 0))],
        out_specs=pl.BlockSpec((MM_BLK, 1), lambda i: (i, 0)),
        out_shape=jax.ShapeDtypeStruct((N_PAD, 1), jnp.float32),
    )(acc, g, dis, b, wh, bh)


def kernel(x, edge_index, W1, b1, W2, b2, Wh, bh):
    src = edge_index[0].astype(jnp.int32)
    dst = edge_index[1].astype(jnp.int32)
    n_edges = src.shape[0]
    chunks = -(-n_edges // (NW * CH))
    pad_n = NW * chunks * CH - n_edges
    pad_vals = N_NODES + (jnp.arange(pad_n, dtype=jnp.int32) % (N_PAD - N_NODES))
    src_p = jnp.concatenate([src, pad_vals]).reshape(NW, chunks, CH)
    dst_p = jnp.concatenate([dst, pad_vals]).reshape(NW, chunks, CH)
    xp = jnp.pad(x, ((0, N_PAD - x.shape[0]), (0, 0)))

    onesD = jnp.zeros((CH, D), jnp.float32).at[:, 0].set(1.0)
    zerosD = jnp.zeros((N_PAD, D), jnp.float32)

    hist = _sc_degree_hist(dst_p, onesD, zerosD)
    p1 = _tc_matmul(xp, W1)
    g1, dis = _tc_norm_scale(hist, p1)
    acc1 = _sc_gather_scatter(g1, src_p, dst_p, zerosD)
    g2 = _tc_layer(acc1, g1, dis, b1.reshape(1, D), W2)
    acc2 = _sc_gather_scatter(g2, src_p, dst_p, zerosD)
    out = _tc_final(acc2, g2, dis, b2.reshape(1, D),
                    Wh.reshape(D, 1), bh.reshape(1, 1))
    return out[:N_NODES]
